# baseline (speedup 1.0000x reference)
import functools

import jax
import jax.numpy as jnp
from jax import lax
from jax.experimental import pallas as pl
from jax.experimental.pallas import tpu as pltpu
from jax.experimental.pallas import tpu_sc as plsc

N_NODES = 10000
N_PAD = 10240
D = 128
E = 320000
NC = 2
NS = 16
NW = NC * NS
K = 128
ROWS = E // K
TR = N_PAD // NS


HALF = N_PAD // NC
ACC_R = HALF + 128
ZR = ACC_R // NS
XR = HALF // NS
ROWS_P = 2528
CPT = ROWS_P // NS
PAIRS = CPT // 2


def _sc_agg_body(ei3, y, zeros, out, sdA, idxA, rowsA, sdB, idxB, rowsB,
                 acc, gA, gB):
    c = lax.axis_index("c")
    s = lax.axis_index("s")

    pltpu.sync_copy(zeros, acc.at[pl.ds(s * ZR, ZR)])
    plsc.subcore_barrier()

    base = c * HALF

    def remap(sd, idx):
        for q in range(K // 16):
            d = sd[1, pl.ds(q * 16, 16)] - base
            ok = (d >= 0) & (d < HALF)
            idx[pl.ds(q * 16, 16)] = jnp.where(ok, d, HALF)

    @pl.loop(0, PAIRS)
    def _pair(g):
        jA = s + NS * 2 * g
        jB = jA + NS
        pltpu.sync_copy(ei3.at[jA], sdA)
        gatA = pltpu.async_copy(y.at[sdA.at[0]], rowsA, gA)
        pltpu.sync_copy(ei3.at[jB], sdB)
        gatB = pltpu.async_copy(y.at[sdB.at[0]], rowsB, gB)
        remap(sdA, idxA)
        gatA.wait()
        pltpu.sync_copy(rowsA, acc.at[idxA], add=True)
        remap(sdB, idxB)
        gatB.wait()
        pltpu.sync_copy(rowsB, acc.at[idxB], add=True)

    plsc.subcore_barrier()
    pltpu.sync_copy(acc.at[pl.ds(s * XR, XR)], out.at[c, pl.ds(s * XR, XR)])


_sc_agg = functools.partial(
    pl.kernel,
    out_type=jax.ShapeDtypeStruct((NC, HALF, D), jnp.float32),
    mesh=plsc.VectorSubcoreMesh(
        core_axis_name="c", subcore_axis_name="s", num_cores=NC, num_subcores=NS
    ),
    scratch_types=[
        pltpu.VMEM((2, K), jnp.int32),
        pltpu.VMEM((K,), jnp.int32),
        pltpu.VMEM((K, D), jnp.float32),
        pltpu.VMEM((2, K), jnp.int32),
        pltpu.VMEM((K,), jnp.int32),
        pltpu.VMEM((K, D), jnp.float32),
        pltpu.VMEM_SHARED((ACC_R, D), jnp.float32),
        pltpu.SemaphoreType.DMA,
        pltpu.SemaphoreType.DMA,
    ],
)(_sc_agg_body)


def _mm_body(x_ref, w_ref, o_ref):
    o_ref[...] = jnp.dot(x_ref[...], w_ref[...], preferred_element_type=jnp.float32)


def _matmul(x, w, bn=1024):
    n, d_in = x.shape
    d_out = w.shape[1]
    return pl.pallas_call(
        _mm_body,
        grid=(n // bn,),
        in_specs=[
            pl.BlockSpec((bn, d_in), lambda i: (i, 0)),
            pl.BlockSpec((d_in, d_out), lambda i: (0, 0)),
        ],
        out_specs=pl.BlockSpec((bn, d_out), lambda i: (i, 0)),
        out_shape=jax.ShapeDtypeStruct((n, d_out), jnp.float32),
    )(x, w)


def _fused_body(eps_ref, y_ref, a_ref, ba_ref, wb_ref, bb_ref, wn_ref, o_ref):
    t = (1.0 + eps_ref[0]) * y_ref[...] + a_ref[...] + ba_ref[...]
    t = jnp.maximum(t, 0.0)
    h = jnp.dot(t, wb_ref[...], preferred_element_type=jnp.float32) + bb_ref[...]
    h = jnp.maximum(h, 0.0)
    o_ref[...] = jnp.dot(h, wn_ref[...], preferred_element_type=jnp.float32)


def _final_body(eps_ref, y_ref, a_ref, ba_ref, wb_ref, bb_ref, o_ref):
    t = (1.0 + eps_ref[0]) * y_ref[...] + a_ref[...] + ba_ref[...]
    t = jnp.maximum(t, 0.0)
    o_ref[...] = jnp.dot(t, wb_ref[...], preferred_element_type=jnp.float32) + bb_ref[...]


def _stage_mid(eps, y, a, ba, wb, bb, wn, bn=1024):
    n = y.shape[0]
    row = lambda i: (i, 0)
    fixed = lambda i: (0, 0)
    return pl.pallas_call(
        _fused_body,
        grid=(n // bn,),
        in_specs=[
            pl.BlockSpec(memory_space=pltpu.SMEM),
            pl.BlockSpec((bn, D), row),
            pl.BlockSpec((bn, D), row),
            pl.BlockSpec((1, D), fixed),
            pl.BlockSpec((D, D), fixed),
            pl.BlockSpec((1, D), fixed),
            pl.BlockSpec((D, D), fixed),
        ],
        out_specs=pl.BlockSpec((bn, D), row),
        out_shape=jax.ShapeDtypeStruct((n, D), jnp.float32),
    )(eps, y, a, ba, wb, bb, wn)


def _stage_final(eps, y, a, ba, wb, bb, bn=1024):
    n = y.shape[0]
    row = lambda i: (i, 0)
    fixed = lambda i: (0, 0)
    return pl.pallas_call(
        _final_body,
        grid=(n // bn,),
        in_specs=[
            pl.BlockSpec(memory_space=pltpu.SMEM),
            pl.BlockSpec((bn, D), row),
            pl.BlockSpec((bn, D), row),
            pl.BlockSpec((1, D), fixed),
            pl.BlockSpec((D, D), fixed),
            pl.BlockSpec((1, D), fixed),
        ],
        out_specs=pl.BlockSpec((bn, D), row),
        out_shape=jax.ShapeDtypeStruct((n, D), jnp.float32),
    )(eps, y, a, ba, wb, bb)


def kernel(x, edge_index, eps1, W1a, b1a, W1b, b1b, eps2, W2a, b2a, W2b, b2b):
    ei = edge_index.astype(jnp.int32)
    pad_e = ROWS_P * K - E
    eip = jnp.stack([jnp.pad(ei[0], (0, pad_e)),
                     jnp.pad(ei[1], (0, pad_e), constant_values=N_NODES)])
    ei3 = jnp.transpose(eip.reshape(2, ROWS_P, K), (1, 0, 2))
    zeros = jnp.zeros((ZR, D), jnp.float32)
    e1 = jnp.reshape(eps1, (1,))
    e2 = jnp.reshape(eps2, (1,))
    b1a_ = jnp.reshape(b1a, (1, D))
    b1b_ = jnp.reshape(b1b, (1, D))
    b2a_ = jnp.reshape(b2a, (1, D))
    b2b_ = jnp.reshape(b2b, (1, D))

    xp = jnp.pad(x, ((0, N_PAD - N_NODES), (0, 0)))
    y1 = _matmul(xp, W1a)
    a1 = _sc_agg(ei3, y1, zeros).reshape(N_PAD, D)
    y2 = _stage_mid(e1, y1, a1, b1a_, W1b, b1b_, W2a)
    a2 = _sc_agg(ei3, y2, zeros).reshape(N_PAD, D)
    out = _stage_final(e2, y2, a2, b2a_, W2b, b2b_)
    return out[:N_NODES]


# speedup vs baseline: 1.4318x; 1.4318x over previous
import functools

import jax
import jax.numpy as jnp
from jax import lax
from jax.experimental import pallas as pl
from jax.experimental.pallas import tpu as pltpu
from jax.experimental.pallas import tpu_sc as plsc

N_NODES = 10000
N_PAD = 10240
D = 128
E = 320000
NC = 2
NS = 16
K = 128
ROWS = E // K
HALF = N_PAD // NC
ACC_R = HALF + 128
ZR = ACC_R // NS
XR = HALF // NS


def _sc_agg_body(ei3, y, zeros, out, sdv, idxv, rowsv, acc, gsem):
    c = lax.axis_index("c")
    s = lax.axis_index("s")

    pltpu.sync_copy(zeros, acc.at[pl.ds(s * ZR, ZR)])
    plsc.subcore_barrier()

    base = c * HALF

    @pl.loop(s, ROWS, step=NS)
    def _edges(j):
        pltpu.sync_copy(ei3.at[j], sdv)
        gat = pltpu.async_copy(y.at[sdv.at[0]], rowsv, gsem)
        for q in range(K // 16):
            d = sdv[1, pl.ds(q * 16, 16)] - base
            ok = (d >= 0) & (d < HALF)
            idxv[pl.ds(q * 16, 16)] = jnp.where(ok, d, HALF)
        gat.wait()
        pltpu.sync_copy(rowsv, acc.at[idxv], add=True)

    plsc.subcore_barrier()
    pltpu.sync_copy(acc.at[pl.ds(s * XR, XR)], out.at[c, pl.ds(s * XR, XR)])


_sc_agg = functools.partial(
    pl.kernel,
    out_type=jax.ShapeDtypeStruct((NC, HALF, D), jnp.float32),
    mesh=plsc.VectorSubcoreMesh(
        core_axis_name="c", subcore_axis_name="s", num_cores=NC, num_subcores=NS
    ),
    scratch_types=[
        pltpu.VMEM((2, K), jnp.int32),
        pltpu.VMEM((K,), jnp.int32),
        pltpu.VMEM((K, D), jnp.float32),
        pltpu.VMEM_SHARED((ACC_R, D), jnp.float32),
        pltpu.SemaphoreType.DMA,
    ],
)(_sc_agg_body)


def _mm_body(x_ref, w_ref, o_ref):
    o_ref[...] = jnp.dot(x_ref[...], w_ref[...], preferred_element_type=jnp.float32)


def _matmul(x, w, bn=1024):
    n, d_in = x.shape
    d_out = w.shape[1]
    return pl.pallas_call(
        _mm_body,
        grid=(n // bn,),
        in_specs=[
            pl.BlockSpec((bn, d_in), lambda i: (i, 0)),
            pl.BlockSpec((d_in, d_out), lambda i: (0, 0)),
        ],
        out_specs=pl.BlockSpec((bn, d_out), lambda i: (i, 0)),
        out_shape=jax.ShapeDtypeStruct((n, d_out), jnp.float32),
    )(x, w)


def _fused_body(eps_ref, y_ref, a_ref, ba_ref, wb_ref, bb_ref, wn_ref, o_ref):
    t = (1.0 + eps_ref[0]) * y_ref[...] + a_ref[...] + ba_ref[...]
    t = jnp.maximum(t, 0.0)
    h = jnp.dot(t, wb_ref[...], preferred_element_type=jnp.float32) + bb_ref[...]
    h = jnp.maximum(h, 0.0)
    o_ref[...] = jnp.dot(h, wn_ref[...], preferred_element_type=jnp.float32)


def _final_body(eps_ref, y_ref, a_ref, ba_ref, wb_ref, bb_ref, o_ref):
    t = (1.0 + eps_ref[0]) * y_ref[...] + a_ref[...] + ba_ref[...]
    t = jnp.maximum(t, 0.0)
    o_ref[...] = jnp.dot(t, wb_ref[...], preferred_element_type=jnp.float32) + bb_ref[...]


def _stage_mid(eps, y, a, ba, wb, bb, wn, bn=1024):
    n = y.shape[0]
    row = lambda i: (i, 0)
    fixed = lambda i: (0, 0)
    return pl.pallas_call(
        _fused_body,
        grid=(n // bn,),
        in_specs=[
            pl.BlockSpec(memory_space=pltpu.SMEM),
            pl.BlockSpec((bn, D), row),
            pl.BlockSpec((bn, D), row),
            pl.BlockSpec((1, D), fixed),
            pl.BlockSpec((D, D), fixed),
            pl.BlockSpec((1, D), fixed),
            pl.BlockSpec((D, D), fixed),
        ],
        out_specs=pl.BlockSpec((bn, D), row),
        out_shape=jax.ShapeDtypeStruct((n, D), jnp.float32),
    )(eps, y, a, ba, wb, bb, wn)


def _stage_final(eps, y, a, ba, wb, bb, bn=1024):
    n = y.shape[0]
    row = lambda i: (i, 0)
    fixed = lambda i: (0, 0)
    return pl.pallas_call(
        _final_body,
        grid=(n // bn,),
        in_specs=[
            pl.BlockSpec(memory_space=pltpu.SMEM),
            pl.BlockSpec((bn, D), row),
            pl.BlockSpec((bn, D), row),
            pl.BlockSpec((1, D), fixed),
            pl.BlockSpec((D, D), fixed),
            pl.BlockSpec((1, D), fixed),
        ],
        out_specs=pl.BlockSpec((bn, D), row),
        out_shape=jax.ShapeDtypeStruct((n, D), jnp.float32),
    )(eps, y, a, ba, wb, bb)


def kernel(x, edge_index, eps1, W1a, b1a, W1b, b1b, eps2, W2a, b2a, W2b, b2b):
    ei = edge_index.astype(jnp.int32)
    ei3 = jnp.transpose(ei.reshape(2, ROWS, K), (1, 0, 2))
    zeros = jnp.zeros((ZR, D), jnp.float32)
    e1 = jnp.reshape(eps1, (1,))
    e2 = jnp.reshape(eps2, (1,))
    b1a_ = jnp.reshape(b1a, (1, D))
    b1b_ = jnp.reshape(b1b, (1, D))
    b2a_ = jnp.reshape(b2a, (1, D))
    b2b_ = jnp.reshape(b2b, (1, D))

    xp = jnp.pad(x, ((0, N_PAD - N_NODES), (0, 0)))
    y1 = _matmul(xp, W1a)
    a1 = _sc_agg(ei3, y1, zeros).reshape(N_PAD, D)
    y2 = _stage_mid(e1, y1, a1, b1a_, W1b, b1b_, W2a)
    a2 = _sc_agg(ei3, y2, zeros).reshape(N_PAD, D)
    out = _stage_final(e2, y2, a2, b2a_, W2b, b2b_)
    return out[:N_NODES]
